# baseline (speedup 1.0000x reference)
import jax
import jax.numpy as jnp
from jax import lax
from jax.experimental import pallas as pl
from jax.experimental.pallas import tpu as pltpu
from jax.experimental.pallas import tpu_sc as plsc

NC = 2
NS = 16
TILES = NC * NS
CH = 96
CPY = 80
BLK = 512


def _pad_rows(n, m):
  return ((n + m - 1) // m) * m


def _scatter_kernel(nch, npad, d, gather):
  rows = npad // NS
  ncopy = rows // CPY

  assert nch % 2 == 0 and rows % CPY == 0

  def body(g_hbm, srcr_hbm, dstr_hbm, zrows_hbm, out_hbm,
           src_v, dst_v, buf0_v, buf1_v, agg_sh, sem0, sem1, semz):
    c = lax.axis_index("c")
    s = lax.axis_index("s")
    wid = s * NC + c
    if gather:
      pltpu.async_copy(srcr_hbm.at[wid], src_v, sem0)
    pltpu.async_copy(dstr_hbm.at[wid], dst_v, sem1)
    zsrc = buf0_v.at[pl.ds(0, CPY)]
    pltpu.sync_copy(zrows_hbm, zsrc)
    for r in range(ncopy):
      pltpu.async_copy(zsrc, agg_sh.at[pl.ds(s * rows + r * CPY, CPY)], semz)
    for r in range(ncopy):
      pltpu.make_async_copy(
          zsrc, agg_sh.at[pl.ds(s * rows + r * CPY, CPY)], semz).wait()
    if gather:
      pltpu.make_async_copy(srcr_hbm.at[wid], src_v, sem0).wait()
    pltpu.make_async_copy(dstr_hbm.at[wid], dst_v, sem1).wait()
    if not gather:
      pltpu.sync_copy(g_hbm.at[pl.ds(0, CH)], buf0_v)
    plsc.subcore_barrier()

    if gather:
      def sidx(j):
        return src_v.at[pl.ds(j * CH, CH)]

      pltpu.async_copy(g_hbm.at[sidx(0)], buf0_v, sem0)

      def step(i, carry):
        j0 = 2 * i
        j1 = j0 + 1
        j2 = jnp.minimum(j0 + 2, nch - 1)
        pltpu.make_async_copy(g_hbm.at[sidx(j0)], buf0_v, sem0).wait()
        pltpu.async_copy(g_hbm.at[sidx(j1)], buf1_v, sem1)
        pltpu.sync_copy(buf0_v, agg_sh.at[dst_v.at[j0]], add=True)
        pltpu.make_async_copy(g_hbm.at[sidx(j1)], buf1_v, sem1).wait()
        pltpu.async_copy(g_hbm.at[sidx(j2)], buf0_v, sem0)
        pltpu.sync_copy(buf1_v, agg_sh.at[dst_v.at[j1]], add=True)
        return carry

      lax.fori_loop(0, nch // 2, step, 0)
      pltpu.make_async_copy(g_hbm.at[sidx(0)], buf0_v, sem0).wait()
    else:

      def step(j, carry):
        pltpu.sync_copy(buf0_v, agg_sh.at[dst_v.at[j]], add=True)
        return carry

      lax.fori_loop(0, nch, step, 0)
    plsc.subcore_barrier()
    bufs = [buf0_v.at[pl.ds(0, CPY)], buf1_v.at[pl.ds(0, CPY)]]
    sems = [sem0, sem1]
    pltpu.sync_copy(agg_sh.at[pl.ds(s * rows, CPY)], bufs[0])
    for r in range(ncopy):
      cur = bufs[r % 2]
      csem = sems[r % 2]
      dst_slice = out_hbm.at[pl.ds(c * npad + s * rows + r * CPY, CPY)]
      pltpu.async_copy(cur, dst_slice, csem)
      if r + 1 < ncopy:
        pltpu.sync_copy(agg_sh.at[pl.ds(s * rows + (r + 1) * CPY, CPY)],
                        bufs[(r + 1) % 2])
      pltpu.make_async_copy(cur, dst_slice, csem).wait()

  return pl.kernel(
      body,
      out_type=jax.ShapeDtypeStruct((NC * npad, d), jnp.float32),
      mesh=plsc.VectorSubcoreMesh(core_axis_name="c", subcore_axis_name="s"),
      scratch_types=[
          pltpu.VMEM((nch * CH,), jnp.int32),
          pltpu.VMEM((nch, CH), jnp.int32),
          pltpu.VMEM((CH, d), jnp.float32),
          pltpu.VMEM((CH, d), jnp.float32),
          pltpu.VMEM_SHARED((npad, d), jnp.float32),
          pltpu.SemaphoreType.DMA,
          pltpu.SemaphoreType.DMA,
          pltpu.SemaphoreType.DMA,
      ],
  )


def _stage0_body(x_ref, w_ref, d0_ref, d1_ref, g_ref, dv_ref):
  deg = d0_ref[:, 0:1] + d1_ref[:, 0:1]
  dinv = lax.rsqrt(deg + 1.0)
  dv = jnp.broadcast_to(dinv, (BLK, 128))
  dv_ref[...] = dv
  h1 = jnp.dot(x_ref[...], w_ref[...], preferred_element_type=jnp.float32)
  g_ref[...] = dv * h1


def _stage_mid_body(p0_ref, p1_ref, g_ref, dv_ref, bc_ref,
                    w1_ref, b1_ref, w2_ref, b2_ref, wn_ref, gn_ref):
  dv = dv_ref[...]
  conv = dv * (p0_ref[...] + p1_ref[...] + g_ref[...]) + bc_ref[...]
  a = jnp.maximum(
      jnp.dot(conv, w1_ref[...], preferred_element_type=jnp.float32)
      + b1_ref[...], 0.0)
  h = jnp.maximum(
      jnp.dot(a, w2_ref[...], preferred_element_type=jnp.float32)
      + b2_ref[...], 0.0)
  gn_ref[...] = dv * jnp.dot(h, wn_ref[...], preferred_element_type=jnp.float32)


def _stage3_body(p0_ref, p1_ref, g_ref, dv_ref, bc_ref,
                 w1_ref, b1_ref, w2_ref, b2_ref,
                 f1_ref, f1b_ref, f2_ref, f2b_ref, y_ref):
  conv = dv_ref[...] * (p0_ref[...] + p1_ref[...] + g_ref[...]) + bc_ref[...]
  a = jnp.maximum(
      jnp.dot(conv, w1_ref[...], preferred_element_type=jnp.float32)
      + b1_ref[...], 0.0)
  h = jnp.maximum(
      jnp.dot(a, w2_ref[...], preferred_element_type=jnp.float32)
      + b2_ref[...], 0.0)
  z = jnp.maximum(
      jnp.dot(h, f1_ref[...], preferred_element_type=jnp.float32)
      + f1b_ref[...], 0.0)
  t = jnp.dot(z, f2_ref[...], preferred_element_type=jnp.float32) + f2b_ref[...]
  y_ref[...] = 1.0 / (1.0 + jnp.exp(-t))


def _row_spec(d):
  return pl.BlockSpec((BLK, d), lambda i: (i, 0))


def _full_spec(r, c):
  return pl.BlockSpec((r, c), lambda i: (0, 0))


def _call_stage0(xp, wc0, deg0, deg1, npad):
  grid = npad // BLK
  return pl.pallas_call(
      _stage0_body,
      grid=(grid,),
      in_specs=[_row_spec(128), _full_spec(128, 128), _row_spec(128),
                _row_spec(128)],
      out_specs=(_row_spec(128), _row_spec(128)),
      out_shape=(jax.ShapeDtypeStruct((npad, 128), jnp.float32),
                 jax.ShapeDtypeStruct((npad, 128), jnp.float32)),
  )(xp, wc0, deg0, deg1)


def _call_stage_mid(p0, p1, g, dv, bc, w1, b1, w2, b2, wn, npad):
  grid = npad // BLK
  return pl.pallas_call(
      _stage_mid_body,
      grid=(grid,),
      in_specs=[_row_spec(128), _row_spec(128), _row_spec(128), _row_spec(128),
                _full_spec(1, 128), _full_spec(128, 128), _full_spec(1, 128),
                _full_spec(128, 128), _full_spec(1, 128), _full_spec(128, 128)],
      out_specs=_row_spec(128),
      out_shape=jax.ShapeDtypeStruct((npad, 128), jnp.float32),
  )(p0, p1, g, dv, bc, w1, b1, w2, b2, wn)


def _call_stage3(p0, p1, g, dv, bc, w1, b1, w2, b2, f1, f1b, f2, f2b, npad):
  grid = npad // BLK
  return pl.pallas_call(
      _stage3_body,
      grid=(grid,),
      in_specs=[_row_spec(128), _row_spec(128), _row_spec(128), _row_spec(128),
                _full_spec(1, 128), _full_spec(128, 128), _full_spec(1, 128),
                _full_spec(128, 128), _full_spec(1, 128),
                _full_spec(128, 128), _full_spec(1, 128),
                _full_spec(128, 128), _full_spec(1, 128)],
      out_specs=_row_spec(128),
      out_shape=jax.ShapeDtypeStruct((npad, 128), jnp.float32),
  )(p0, p1, g, dv, bc, w1, b1, w2, b2, f1, f1b, f2, f2b)


def kernel(x, edge_index, Wc0, bc0, m0w1, m0b1, m0w2, m0b2,
           Wc1, bc1, m1w1, m1b1, m1w2, m1b2,
           Wc2, bc2, m2w1, m2b1, m2w2, m2b2,
           fc1_w, fc1_b, fc2_w, fc2_b):
  n, d = x.shape
  e = edge_index.shape[1]
  h2 = fc1_w.shape[1]

  npad = _pad_rows(n + 1, 2048)
  npad = _pad_rows(npad, BLK)

  nch = (e + TILES * CH - 1) // (TILES * CH)
  nch = nch + (nch % 2)
  epad = TILES * CH * nch

  src = edge_index[0]
  dst = edge_index[1]
  pad_e = epad - e
  srcp = jnp.concatenate([src, jnp.zeros((pad_e,), jnp.int32)])
  dstp = jnp.concatenate([dst, jnp.full((pad_e,), n, jnp.int32)])
  srcr = srcp.reshape(TILES, nch * CH)
  dstr = dstp.reshape(TILES, nch, CH)

  xp = jnp.zeros((npad, d), jnp.float32).at[:n].set(x)
  ones_t = jnp.ones((CH, d), jnp.float32)
  z128 = jnp.zeros((CPY, d), jnp.float32)

  count = _scatter_kernel(nch, npad, d, gather=False)
  scat = _scatter_kernel(nch, npad, d, gather=True)

  degp = count(ones_t, srcr, dstr, z128)
  g0, dv = _call_stage0(xp, Wc0, degp[:npad], degp[npad:], npad)

  def row(v):
    return v.reshape(1, -1)

  p = scat(g0, srcr, dstr, z128)
  g1 = _call_stage_mid(p[:npad], p[npad:], g0, dv, row(bc0), m0w1, row(m0b1),
                       m0w2, row(m0b2), Wc1, npad)
  p = scat(g1, srcr, dstr, z128)
  g2 = _call_stage_mid(p[:npad], p[npad:], g1, dv, row(bc1), m1w1, row(m1b1),
                       m1w2, row(m1b2), Wc2, npad)
  p = scat(g2, srcr, dstr, z128)

  f1 = jnp.zeros((d, d), jnp.float32).at[:, :h2].set(fc1_w)
  f1b = jnp.zeros((1, d), jnp.float32).at[0, :h2].set(fc1_b)
  f2 = jnp.zeros((d, d), jnp.float32).at[:h2, 0].set(fc2_w[:, 0])
  f2b = jnp.zeros((1, d), jnp.float32).at[0, 0].set(fc2_b[0])

  y = _call_stage3(p[:npad], p[npad:], g2, dv, row(bc2), m2w1, row(m2b1),
                   m2w2, row(m2b2), f1, f1b, f2, f2b, npad)
  return y[:n, :1]


# speedup vs baseline: 1.5484x; 1.5484x over previous
import jax
import jax.numpy as jnp
from jax import lax
from jax.experimental import pallas as pl
from jax.experimental.pallas import tpu as pltpu
from jax.experimental.pallas import tpu_sc as plsc

NC = 2
NS = 16
TILES = NC * NS
CH = 80
BLK = 512


def _pad_rows(n, m):
  return ((n + m - 1) // m) * m


def _scatter_kernel(nch, npad, d, gather):
  rows = npad // NS
  ncopy = rows // CH

  assert nch % 2 == 0

  def body(g_hbm, srcr_hbm, dstr_hbm, zrows_hbm, out_hbm,
           src_v, dst_v, buf0_v, buf1_v, agg_sh, sem0, sem1, semz):
    c = lax.axis_index("c")
    s = lax.axis_index("s")
    wid = s * NC + c
    if gather:
      pltpu.async_copy(srcr_hbm.at[wid], src_v, sem0)
    pltpu.async_copy(dstr_hbm.at[wid], dst_v, sem1)
    pltpu.sync_copy(zrows_hbm, buf0_v)
    for r in range(ncopy):
      pltpu.async_copy(buf0_v, agg_sh.at[pl.ds(s * rows + r * CH, CH)], semz)
    for r in range(ncopy):
      pltpu.make_async_copy(
          buf0_v, agg_sh.at[pl.ds(s * rows + r * CH, CH)], semz).wait()
    if gather:
      pltpu.make_async_copy(srcr_hbm.at[wid], src_v, sem0).wait()
    pltpu.make_async_copy(dstr_hbm.at[wid], dst_v, sem1).wait()
    if not gather:
      pltpu.sync_copy(g_hbm.at[pl.ds(0, CH)], buf0_v)
    plsc.subcore_barrier()

    if gather:
      def sidx(j):
        return src_v.at[pl.ds(j * CH, CH)]

      pltpu.async_copy(g_hbm.at[sidx(0)], buf0_v, sem0)

      def step(i, carry):
        j0 = 2 * i
        j1 = j0 + 1
        j2 = jnp.minimum(j0 + 2, nch - 1)
        pltpu.make_async_copy(g_hbm.at[sidx(j0)], buf0_v, sem0).wait()
        pltpu.async_copy(g_hbm.at[sidx(j1)], buf1_v, sem1)
        pltpu.sync_copy(buf0_v, agg_sh.at[dst_v.at[j0]], add=True)
        pltpu.make_async_copy(g_hbm.at[sidx(j1)], buf1_v, sem1).wait()
        pltpu.async_copy(g_hbm.at[sidx(j2)], buf0_v, sem0)
        pltpu.sync_copy(buf1_v, agg_sh.at[dst_v.at[j1]], add=True)
        return carry

      lax.fori_loop(0, nch // 2, step, 0)
      pltpu.make_async_copy(g_hbm.at[sidx(0)], buf0_v, sem0).wait()
    else:

      def step(j, carry):
        pltpu.sync_copy(buf0_v, agg_sh.at[dst_v.at[j]], add=True)
        return carry

      lax.fori_loop(0, nch, step, 0)
    plsc.subcore_barrier()
    bufs = [buf0_v, buf1_v]
    sems = [sem0, sem1]
    pltpu.sync_copy(agg_sh.at[pl.ds(s * rows, CH)], buf0_v)
    for r in range(ncopy):
      cur = bufs[r % 2]
      csem = sems[r % 2]
      dst_slice = out_hbm.at[pl.ds(c * npad + s * rows + r * CH, CH)]
      pltpu.async_copy(cur, dst_slice, csem)
      if r + 1 < ncopy:
        pltpu.sync_copy(agg_sh.at[pl.ds(s * rows + (r + 1) * CH, CH)],
                        bufs[(r + 1) % 2])
      pltpu.make_async_copy(cur, dst_slice, csem).wait()

  return pl.kernel(
      body,
      out_type=jax.ShapeDtypeStruct((NC * npad, d), jnp.float32),
      mesh=plsc.VectorSubcoreMesh(core_axis_name="c", subcore_axis_name="s"),
      scratch_types=[
          pltpu.VMEM((nch * CH,), jnp.int32),
          pltpu.VMEM((nch, CH), jnp.int32),
          pltpu.VMEM((CH, d), jnp.float32),
          pltpu.VMEM((CH, d), jnp.float32),
          pltpu.VMEM_SHARED((npad, d), jnp.float32),
          pltpu.SemaphoreType.DMA,
          pltpu.SemaphoreType.DMA,
          pltpu.SemaphoreType.DMA,
      ],
  )


def _stage0_body(x_ref, w_ref, d0_ref, d1_ref, g_ref, dv_ref):
  deg = d0_ref[:, 0:1] + d1_ref[:, 0:1]
  dinv = lax.rsqrt(deg + 1.0)
  dv = jnp.broadcast_to(dinv, (BLK, 128))
  dv_ref[...] = dv
  h1 = jnp.dot(x_ref[...], w_ref[...], preferred_element_type=jnp.float32)
  g_ref[...] = dv * h1


def _stage_mid_body(p0_ref, p1_ref, g_ref, dv_ref, bc_ref,
                    w1_ref, b1_ref, w2_ref, b2_ref, wn_ref, gn_ref):
  dv = dv_ref[...]
  conv = dv * (p0_ref[...] + p1_ref[...] + g_ref[...]) + bc_ref[...]
  a = jnp.maximum(
      jnp.dot(conv, w1_ref[...], preferred_element_type=jnp.float32)
      + b1_ref[...], 0.0)
  h = jnp.maximum(
      jnp.dot(a, w2_ref[...], preferred_element_type=jnp.float32)
      + b2_ref[...], 0.0)
  gn_ref[...] = dv * jnp.dot(h, wn_ref[...], preferred_element_type=jnp.float32)


def _stage3_body(p0_ref, p1_ref, g_ref, dv_ref, bc_ref,
                 w1_ref, b1_ref, w2_ref, b2_ref,
                 f1_ref, f1b_ref, f2_ref, f2b_ref, y_ref):
  conv = dv_ref[...] * (p0_ref[...] + p1_ref[...] + g_ref[...]) + bc_ref[...]
  a = jnp.maximum(
      jnp.dot(conv, w1_ref[...], preferred_element_type=jnp.float32)
      + b1_ref[...], 0.0)
  h = jnp.maximum(
      jnp.dot(a, w2_ref[...], preferred_element_type=jnp.float32)
      + b2_ref[...], 0.0)
  z = jnp.maximum(
      jnp.dot(h, f1_ref[...], preferred_element_type=jnp.float32)
      + f1b_ref[...], 0.0)
  t = jnp.dot(z, f2_ref[...], preferred_element_type=jnp.float32) + f2b_ref[...]
  y_ref[...] = 1.0 / (1.0 + jnp.exp(-t))


def _row_spec(d):
  return pl.BlockSpec((BLK, d), lambda i: (i, 0))


def _full_spec(r, c):
  return pl.BlockSpec((r, c), lambda i: (0, 0))


def _call_stage0(xp, wc0, deg0, deg1, npad):
  grid = npad // BLK
  return pl.pallas_call(
      _stage0_body,
      grid=(grid,),
      in_specs=[_row_spec(128), _full_spec(128, 128), _row_spec(128),
                _row_spec(128)],
      out_specs=(_row_spec(128), _row_spec(128)),
      out_shape=(jax.ShapeDtypeStruct((npad, 128), jnp.float32),
                 jax.ShapeDtypeStruct((npad, 128), jnp.float32)),
  )(xp, wc0, deg0, deg1)


def _call_stage_mid(p0, p1, g, dv, bc, w1, b1, w2, b2, wn, npad):
  grid = npad // BLK
  return pl.pallas_call(
      _stage_mid_body,
      grid=(grid,),
      in_specs=[_row_spec(128), _row_spec(128), _row_spec(128), _row_spec(128),
                _full_spec(1, 128), _full_spec(128, 128), _full_spec(1, 128),
                _full_spec(128, 128), _full_spec(1, 128), _full_spec(128, 128)],
      out_specs=_row_spec(128),
      out_shape=jax.ShapeDtypeStruct((npad, 128), jnp.float32),
  )(p0, p1, g, dv, bc, w1, b1, w2, b2, wn)


def _call_stage3(p0, p1, g, dv, bc, w1, b1, w2, b2, f1, f1b, f2, f2b, npad):
  grid = npad // BLK
  return pl.pallas_call(
      _stage3_body,
      grid=(grid,),
      in_specs=[_row_spec(128), _row_spec(128), _row_spec(128), _row_spec(128),
                _full_spec(1, 128), _full_spec(128, 128), _full_spec(1, 128),
                _full_spec(128, 128), _full_spec(1, 128),
                _full_spec(128, 128), _full_spec(1, 128),
                _full_spec(128, 128), _full_spec(1, 128)],
      out_specs=_row_spec(128),
      out_shape=jax.ShapeDtypeStruct((npad, 128), jnp.float32),
  )(p0, p1, g, dv, bc, w1, b1, w2, b2, f1, f1b, f2, f2b)


def kernel(x, edge_index, Wc0, bc0, m0w1, m0b1, m0w2, m0b2,
           Wc1, bc1, m1w1, m1b1, m1w2, m1b2,
           Wc2, bc2, m2w1, m2b1, m2w2, m2b2,
           fc1_w, fc1_b, fc2_w, fc2_b):
  n, d = x.shape
  e = edge_index.shape[1]
  h2 = fc1_w.shape[1]

  npad = _pad_rows(n + 1, 2048)
  npad = _pad_rows(npad, BLK)

  nch = (e + TILES * CH - 1) // (TILES * CH)
  nch = nch + (nch % 2)
  epad = TILES * CH * nch

  src = edge_index[0]
  dst = edge_index[1]
  pad_e = epad - e
  srcp = jnp.concatenate([src, jnp.zeros((pad_e,), jnp.int32)])
  dstp = jnp.concatenate([dst, jnp.full((pad_e,), n, jnp.int32)])
  srcr = srcp.reshape(TILES, nch * CH)
  dstr = dstp.reshape(TILES, nch, CH)

  xp = jnp.zeros((npad, d), jnp.float32).at[:n].set(x)
  ones_t = jnp.ones((CH, d), jnp.float32)
  z128 = jnp.zeros((CH, d), jnp.float32)

  count = _scatter_kernel(nch, npad, d, gather=False)
  scat = _scatter_kernel(nch, npad, d, gather=True)

  degp = count(ones_t, srcr, dstr, z128)
  g0, dv = _call_stage0(xp, Wc0, degp[:npad], degp[npad:], npad)

  def row(v):
    return v.reshape(1, -1)

  p = scat(g0, srcr, dstr, z128)
  g1 = _call_stage_mid(p[:npad], p[npad:], g0, dv, row(bc0), m0w1, row(m0b1),
                       m0w2, row(m0b2), Wc1, npad)
  p = scat(g1, srcr, dstr, z128)
  g2 = _call_stage_mid(p[:npad], p[npad:], g1, dv, row(bc1), m1w1, row(m1b1),
                       m1w2, row(m1b2), Wc2, npad)
  p = scat(g2, srcr, dstr, z128)

  f1 = jnp.zeros((d, d), jnp.float32).at[:, :h2].set(fc1_w)
  f1b = jnp.zeros((1, d), jnp.float32).at[0, :h2].set(fc1_b)
  f2 = jnp.zeros((d, d), jnp.float32).at[:h2, 0].set(fc2_w[:, 0])
  f2b = jnp.zeros((1, d), jnp.float32).at[0, 0].set(fc2_b[0])

  y = _call_stage3(p[:npad], p[npad:], g2, dv, row(bc2), m2w1, row(m2b1),
                   m2w2, row(m2b2), f1, f1b, f2, f2b, npad)
  return y[:n, :1]
